# Initial kernel scaffold; baseline (speedup 1.0000x reference)
import jax
import jax.numpy as jnp
from jax.experimental import pallas as pl


def kernel(x, edge_index, W_l1, b_l1, W_r1, W_l2, b_l2, W_r2, W_l3, b_l3, W_r3):
    raise NotImplementedError("write your pallas kernel here")


# speedup vs baseline: 7.3173x; 7.3173x over previous
import functools

import jax
import jax.numpy as jnp
from jax import lax
from jax.experimental import pallas as pl
from jax.experimental.pallas import tpu as pltpu
from jax.experimental.pallas import tpu_sc as plsc

N_NODES = 10000
D = 128
N_EDGES = 640000

NC = 2
NS = 16
NW = NC * NS
CHUNK = 128
NCHUNKS = 157
E_PAD = NW * NCHUNKS * CHUNK
PAD_ROWS = 10240
ROWS_PER_TILE = 624
TAIL_ROWS = N_NODES - NS * ROWS_PER_TILE
DEG_W = 16


def _make_agg():
    mesh = plsc.VectorSubcoreMesh(core_axis_name="c", subcore_axis_name="s")
    out_type = [jax.ShapeDtypeStruct((NC, N_NODES, D), jnp.float32)]
    scratch = [
        pltpu.VMEM_SHARED((PAD_ROWS, D), jnp.float32),
        pltpu.VMEM((2, 2, CHUNK), jnp.int32),
        pltpu.VMEM((2, CHUNK, D), jnp.float32),
        pltpu.SemaphoreType.DMA,
    ]

    @functools.partial(pl.kernel, mesh=mesh, out_type=out_type,
                       scratch_types=scratch)
    def agg_kernel(h_hbm, idx_hbm, agg_out, agg_sh, idx_v, rows_v, sem):
        c = lax.axis_index("c")
        s = lax.axis_index("s")
        w = c * NS + s

        def _zero_row(i, _):
            for j in range(D // 16):
                rows_v[0, i, pl.ds(j * 16, 16)] = jnp.zeros((16,), jnp.float32)
            return 0
        lax.fori_loop(0, CHUNK, _zero_row, 0)
        for k in range(PAD_ROWS // NS // CHUNK):
            pltpu.sync_copy(rows_v.at[0],
                            agg_sh.at[pl.ds(s * (PAD_ROWS // NS) + k * CHUNK, CHUNK)])
        plsc.subcore_barrier()

        def _body(g, _):
            pltpu.sync_copy(idx_hbm.at[w, g], idx_v.at[0])
            pltpu.async_copy(h_hbm.at[idx_v.at[0, 0]], rows_v.at[0], sem).wait()
            pltpu.sync_copy(rows_v.at[0], agg_sh.at[idx_v.at[0, 1]], add=True)
            return 0
        lax.fori_loop(0, NCHUNKS, _body, 0)

        plsc.subcore_barrier()

        pltpu.sync_copy(agg_sh.at[pl.ds(s * ROWS_PER_TILE, ROWS_PER_TILE)],
                        agg_out.at[c, pl.ds(s * ROWS_PER_TILE, ROWS_PER_TILE)])
        @pl.when(s == NS - 1)
        def _():
            pltpu.sync_copy(agg_sh.at[pl.ds(NS * ROWS_PER_TILE, TAIL_ROWS)],
                            agg_out.at[c, pl.ds(NS * ROWS_PER_TILE, TAIL_ROWS)])

    return agg_kernel


_BINS_PER_TILE = PAD_ROWS // NS


def _make_deg():
    mesh = plsc.VectorSubcoreMesh(core_axis_name="c", subcore_axis_name="s")
    out_type = [jax.ShapeDtypeStruct((NC, 1, PAD_ROWS), jnp.float32)]
    scratch = [
        pltpu.VMEM_SHARED((NS, PAD_ROWS), jnp.float32),
        pltpu.VMEM((2, 2, CHUNK), jnp.int32),
        pltpu.VMEM((PAD_ROWS,), jnp.float32),
        pltpu.VMEM((NS, _BINS_PER_TILE), jnp.float32),
    ]

    @functools.partial(pl.kernel, mesh=mesh, out_type=out_type,
                       scratch_types=scratch,
                       compiler_params=pltpu.CompilerParams(
                           needs_layout_passes=False))
    def deg_kernel(idx_hbm, deg_out, deg_sh, idx_v, hist_v, stripe_v):
        c = lax.axis_index("c")
        s = lax.axis_index("s")
        w = c * NS + s

        def _zero(i, _):
            hist_v[pl.ds(i * 16, 16)] = jnp.zeros((16,), jnp.float32)
            return 0
        lax.fori_loop(0, PAD_ROWS // 16, _zero, 0)

        ones16 = jnp.ones((16,), jnp.float32)

        def _body(g, _):
            pltpu.sync_copy(idx_hbm.at[w, g], idx_v.at[0])
            for j in range(CHUNK // 16):
                d = idx_v[0, 1, pl.ds(j * 16, 16)]
                plsc.addupdate_scatter(hist_v, [d], ones16)
            return 0
        lax.fori_loop(0, NCHUNKS, _body, 0)

        pltpu.sync_copy(hist_v, deg_sh.at[s])
        plsc.subcore_barrier()

        for r in range(NS):
            pltpu.sync_copy(deg_sh.at[r, pl.ds(s * _BINS_PER_TILE, _BINS_PER_TILE)],
                            stripe_v.at[r])

        def _reduce(j, _):
            acc = stripe_v[0, pl.ds(j * 16, 16)]
            for r in range(1, NS):
                acc = acc + stripe_v[r, pl.ds(j * 16, 16)]
            stripe_v[0, pl.ds(j * 16, 16)] = acc
            return 0
        lax.fori_loop(0, _BINS_PER_TILE // 16, _reduce, 0)

        pltpu.sync_copy(stripe_v.at[0],
                        deg_out.at[c, 0, pl.ds(s * _BINS_PER_TILE, _BINS_PER_TILE)])

    return deg_kernel


_agg_call = _make_agg()
_deg_call = _make_deg()


def _dense_body(agg_ref, deg_ref, h_ref, wl_ref, wr_ref, b_ref, out_ref):
    deg = jnp.maximum(deg_ref[0] + deg_ref[1], 1.0)
    mean = (agg_ref[0] + agg_ref[1]) / deg
    z = (jnp.dot(mean, wl_ref[...], preferred_element_type=jnp.float32)
         + jnp.dot(h_ref[...], wr_ref[...], preferred_element_type=jnp.float32)
         + b_ref[...])
    out_ref[...] = jnp.where(z > 0, z, jnp.exp(jnp.minimum(z, 0.0)) - 1.0)


_BLK = 1000


def _dense(agg, deg, h, wl, wr, b):
    grid = (N_NODES // _BLK,)
    return pl.pallas_call(
        _dense_body,
        grid=grid,
        in_specs=[
            pl.BlockSpec((NC, _BLK, D), lambda i: (0, i, 0)),
            pl.BlockSpec((NC, _BLK, 1), lambda i: (0, i, 0)),
            pl.BlockSpec((_BLK, D), lambda i: (i, 0)),
            pl.BlockSpec((D, D), lambda i: (0, 0)),
            pl.BlockSpec((D, D), lambda i: (0, 0)),
            pl.BlockSpec((1, D), lambda i: (0, 0)),
        ],
        out_specs=pl.BlockSpec((_BLK, D), lambda i: (i, 0)),
        out_shape=jax.ShapeDtypeStruct((N_NODES, D), jnp.float32),
    )(agg, deg, h, wl, wr, b)


def kernel(x, edge_index, W_l1, b_l1, W_r1, W_l2, b_l2, W_r2, W_l3, b_l3, W_r3):
    src = edge_index[0].astype(jnp.int32)
    dst = edge_index[1].astype(jnp.int32)
    pad = E_PAD - N_EDGES
    src_p = jnp.concatenate([src, jnp.zeros((pad,), jnp.int32)])
    dst_p = jnp.concatenate([dst, jnp.full((pad,), N_NODES, jnp.int32)])
    idx = jnp.stack([src_p.reshape(NW, NCHUNKS, CHUNK),
                     dst_p.reshape(NW, NCHUNKS, CHUNK)], axis=2)

    (deg_pad,) = _deg_call(idx)
    deg = deg_pad[:, 0, :N_NODES].reshape(NC, N_NODES, 1)
    (agg1,) = _agg_call(x, idx)
    h1 = _dense(agg1, deg, x, W_l1.T, W_r1.T, b_l1.reshape(1, D))
    (agg2,) = _agg_call(h1, idx)
    h2 = _dense(agg2, deg, h1, W_l2.T, W_r2.T, b_l2.reshape(1, D))
    (agg3,) = _agg_call(h2, idx)
    h3 = _dense(agg3, deg, h2, W_l3.T, W_r3.T, b_l3.reshape(1, D))
    return h3
